# Initial kernel scaffold; baseline (speedup 1.0000x reference)
import jax
import jax.numpy as jnp
from jax.experimental import pallas as pl


def kernel(adj_indices, adj_values, conditions_values, W_pc1, b_pc1, W_pc2, b_pc2, W_cu1, b_cu1, W_cu2, b_cu2, W_vu1, b_vu1, W_vu2, b_vu2, W_o1, b_o1, W_o2, b_o2):
    raise NotImplementedError("write your pallas kernel here")


# speedup vs baseline: 18.2369x; 18.2369x over previous
import functools

import jax
import jax.numpy as jnp
from jax import lax
from jax.experimental import pallas as pl
from jax.experimental.pallas import tpu as pltpu
from jax.experimental.pallas import tpu_sc as plsc

N = 65536
F = 64
NC = 2
NS = 16
BLK = 128
SUB = 8
SUPER = BLK * SUB
ROWS_PER_TILE = N // NS


def _sc_mesh():
    return plsc.VectorSubcoreMesh(
        core_axis_name="c", subcore_axis_name="s", num_cores=NC, num_subcores=NS)


def _stage_a(cols2d, vals2d, E):
    sbt = (E // 2) // SUPER // NS

    @functools.partial(
        pl.kernel,
        out_type=jax.ShapeDtypeStruct((NC, N, 16), jnp.float32),
        mesh=_sc_mesh(),
        scratch_types=[
            pltpu.VMEM_SHARED((N, 16), jnp.float32),
            pltpu.VMEM((SUB, BLK), jnp.int32),
            pltpu.VMEM((SUB, BLK), jnp.float32),
            pltpu.VMEM((SUPER, 16), jnp.float32),
            pltpu.VMEM((SUPER, 16), jnp.float32),
            pltpu.VMEM((BLK, 16), jnp.float32),
            pltpu.SemaphoreType.DMA,
        ],
        compiler_params=pltpu.CompilerParams(use_tc_tiling_on_sc=False),
    )
    def k(cols_hbm, vals_hbm, out_hbm, acc, cbuf, vbuf, obuf, sbuf, zbuf, ssem):
        c = lax.axis_index("c")
        sid = lax.axis_index("s")
        zero16 = jnp.zeros((16,), jnp.float32)
        one16 = jnp.ones((16,), jnp.float32)

        def ob(i, _):
            obuf[i, :] = one16
            return 0
        lax.fori_loop(0, SUPER, ob, 0)

        def zb(i, _):
            zbuf[i, :] = zero16
            return 0
        lax.fori_loop(0, BLK, zb, 0)

        def zacc(t, _):
            off = pl.multiple_of(sid * ROWS_PER_TILE + t * BLK, 8)
            pltpu.sync_copy(zbuf, acc.at[pl.ds(off, BLK)])
            return 0
        lax.fori_loop(0, ROWS_PER_TILE // BLK, zacc, 0)
        plsc.subcore_barrier()

        base = (c * (E // 2) + sid * sbt * SUPER) // BLK

        def body(sb, _):
            rb = pl.multiple_of(base + sb * SUB, 8)
            pltpu.sync_copy(cols_hbm.at[pl.ds(rb, SUB)], cbuf)
            pltpu.sync_copy(vals_hbm.at[pl.ds(rb, SUB)], vbuf)
            for k8 in range(SUB):
                def bcast(j8, _, k8=k8):
                    va16 = jnp.abs(vbuf[k8, pl.ds(j8 * 16, 16)])
                    e0 = k8 * BLK + j8 * 16
                    for i in range(16):
                        er = e0 + i
                        sbuf[er, :] = obuf[er, :] * va16[i]
                    return 0
                lax.fori_loop(0, BLK // 16, bcast, 0)
            ds = [pltpu.async_copy(sbuf.at[pl.ds(k8 * BLK, BLK)],
                                   acc.at[cbuf.at[k8]], ssem, add=True)
                  for k8 in range(SUB)]
            for d in ds:
                d.wait()
            return 0
        lax.fori_loop(0, sbt, body, 0)
        plsc.subcore_barrier()
        sl = pl.ds(pl.multiple_of(sid * ROWS_PER_TILE, 8), ROWS_PER_TILE)
        pltpu.sync_copy(acc.at[sl], out_hbm.at[c, sl])

    return k(cols2d, vals2d)


def _stage_b(cond, s_part, W_pc1, b_pc1, W_pc2, b_pc2, W_cu1, b_cu1, W_cu2, b_cu2):
    BN = 4096
    grid = N // BN

    def body(cond_ref, s0_ref, s1_ref, wp1, bp1, wp2, bp2, wc1, bc1, wc2, bc2, o_ref):
        cond_b = cond_ref[...]
        s = s0_ref[...] + s1_ref[...]
        h1 = jnp.maximum(cond_b * wp1[...] + bp1[...][None, :], 0.0)
        c0 = jnp.dot(h1, wp2[...], preferred_element_type=jnp.float32) + bp2[...][None, :]
        wc1v = wc1[...]
        wsum = jnp.sum(wc1v[F:, :], axis=0)
        h2 = jnp.maximum(
            jnp.dot(c0, wc1v[:F, :], preferred_element_type=jnp.float32)
            + s * wsum[None, :] + bc1[...][None, :], 0.0)
        cc = jnp.dot(h2, wc2[...], preferred_element_type=jnp.float32) + bc2[...][None, :]
        for g in range(4):
            o_ref[g] = cc[:, g * 16:(g + 1) * 16]

    full = lambda shp: pl.BlockSpec(shp, lambda i: tuple(0 for _ in shp))
    return pl.pallas_call(
        body,
        grid=(grid,),
        in_specs=[
            pl.BlockSpec((BN, 1), lambda i: (i, 0)),
            pl.BlockSpec((BN, 1), lambda i: (i, 0)),
            pl.BlockSpec((BN, 1), lambda i: (i, 0)),
            full((1, 2 * F)), full((2 * F,)),
            full((2 * F, F)), full((F,)),
            full((2 * F, 2 * F)), full((2 * F,)),
            full((2 * F, F)), full((F,)),
        ],
        out_specs=pl.BlockSpec((4, BN, 16), lambda i: (0, i, 0)),
        out_shape=jax.ShapeDtypeStruct((4, N, 16), jnp.float32),
    )(cond.reshape(N, 1), s_part[0].reshape(N, 1), s_part[1].reshape(N, 1),
      W_pc1, b_pc1, W_pc2, b_pc2, W_cu1, b_cu1, W_cu2, b_cu2)


def _stage_c(tab_flat, cols4, rows2d, vals2d, E):
    sbt = E // SUPER // NS

    @functools.partial(
        pl.kernel,
        out_type=jax.ShapeDtypeStruct((4, N, 16), jnp.float32),
        mesh=_sc_mesh(),
        scratch_types=[
            pltpu.VMEM_SHARED((N, 16), jnp.float32),
            pltpu.VMEM((SUB, BLK), jnp.int32),
            pltpu.VMEM((SUB, BLK), jnp.int32),
            pltpu.VMEM((SUB, BLK), jnp.float32),
            pltpu.VMEM((SUPER, 16), jnp.float32),
            pltpu.VMEM((BLK, 16), jnp.float32),
            pltpu.SemaphoreType.DMA,
            pltpu.SemaphoreType.DMA,
        ],
        compiler_params=pltpu.CompilerParams(use_tc_tiling_on_sc=False),
    )
    def k(tab_hbm, cols_hbm, rows_hbm, vals_hbm, out_hbm,
          acc, rbuf, cbuf, vbuf, gbuf, zbuf, gsem, ssem):
        c = lax.axis_index("c")
        sid = lax.axis_index("s")
        zero16 = jnp.zeros((16,), jnp.float32)

        def zb(i, _):
            zbuf[i, :] = zero16
            return 0
        lax.fori_loop(0, BLK, zb, 0)
        acc_slice = pl.ds(pl.multiple_of(sid * ROWS_PER_TILE, 8), ROWS_PER_TILE)
        base = sid * sbt * SUB

        for p in range(2):
            g = NC * c + p

            def zacc(t, _):
                off = pl.multiple_of(sid * ROWS_PER_TILE + t * BLK, 8)
                pltpu.sync_copy(zbuf, acc.at[pl.ds(off, BLK)])
                return 0
            lax.fori_loop(0, ROWS_PER_TILE // BLK, zacc, 0)
            plsc.subcore_barrier()

            def body(sb, _):
                rb = pl.multiple_of(base + sb * SUB, 8)
                pltpu.sync_copy(rows_hbm.at[pl.ds(rb, SUB)], rbuf)
                pltpu.sync_copy(cols_hbm.at[g, pl.ds(rb, SUB)], cbuf)
                pltpu.sync_copy(vals_hbm.at[pl.ds(rb, SUB)], vbuf)
                gds = [pltpu.async_copy(tab_hbm.at[cbuf.at[k8]],
                                        gbuf.at[pl.ds(k8 * BLK, BLK)], gsem)
                       for k8 in range(SUB)]
                for d in gds:
                    d.wait()

                for k8 in range(SUB):
                    def scale(j8, _, k8=k8):
                        va16 = jnp.abs(vbuf[k8, pl.ds(j8 * 16, 16)])
                        e0 = k8 * BLK + j8 * 16
                        for i in range(16):
                            er = e0 + i
                            gbuf[er, :] = gbuf[er, :] * va16[i]
                        return 0
                    lax.fori_loop(0, BLK // 16, scale, 0)

                sds = [pltpu.async_copy(gbuf.at[pl.ds(k8 * BLK, BLK)],
                                        acc.at[rbuf.at[k8]], ssem, add=True)
                       for k8 in range(SUB)]
                for d in sds:
                    d.wait()
                return 0
            lax.fori_loop(0, sbt, body, 0)
            plsc.subcore_barrier()
            pltpu.sync_copy(acc.at[acc_slice], out_hbm.at[g, acc_slice])
            plsc.subcore_barrier()

    return k(tab_flat, cols4, rows2d, vals2d)


def _stage_d(c2v4, W_vu1, b_vu1, W_vu2, b_vu2, W_o1, b_o1, W_o2, b_o2):
    BN = 4096
    grid = N // BN

    def body(g0, g1, g2, g3, wv1, bv1, wv2, bv2, wo1, bo1, wo2, bo2, o_ref):
        x = jnp.concatenate([g0[0], g1[0], g2[0], g3[0]], axis=-1)
        wv1v = wv1[...]
        bias1 = bv1[...] + jnp.sum(wv1v[:F, :], axis=0)
        h = jnp.maximum(jnp.dot(x, wv1v[F:, :], preferred_element_type=jnp.float32)
                        + bias1[None, :], 0.0)
        v = jnp.dot(h, wv2[...], preferred_element_type=jnp.float32) + bv2[...][None, :]
        h3 = jnp.maximum(jnp.dot(v, wo1[...], preferred_element_type=jnp.float32)
                         + bo1[...][None, :], 0.0)
        logit = jnp.dot(h3, wo2[...], preferred_element_type=jnp.float32) + bo2[...][None, :]
        o_ref[...] = 1.0 / (1.0 + jnp.exp(-logit))

    full = lambda shp: pl.BlockSpec(shp, lambda i: tuple(0 for _ in shp))
    gspec = lambda g: pl.BlockSpec((1, BN, 16), lambda i, g=g: (g, i, 0))
    return pl.pallas_call(
        body,
        grid=(grid,),
        in_specs=[
            gspec(0), gspec(1), gspec(2), gspec(3),
            full((2 * F, 2 * F)), full((2 * F,)),
            full((2 * F, F)), full((F,)),
            full((F, 2 * F)), full((2 * F,)),
            full((2 * F, 1)), full((1,)),
        ],
        out_specs=pl.BlockSpec((BN, 1), lambda i: (i, 0)),
        out_shape=jax.ShapeDtypeStruct((N, 1), jnp.float32),
    )(c2v4, c2v4, c2v4, c2v4, W_vu1, b_vu1, W_vu2, b_vu2, W_o1, b_o1, W_o2, b_o2)


def kernel(adj_indices, adj_values, conditions_values,
           W_pc1, b_pc1, W_pc2, b_pc2, W_cu1, b_cu1, W_cu2, b_cu2,
           W_vu1, b_vu1, W_vu2, b_vu2, W_o1, b_o1, W_o2, b_o2):
    rows = adj_indices[0].astype(jnp.int32)
    cols = adj_indices[1].astype(jnp.int32)
    vals = adj_values.astype(jnp.float32)
    nnz = rows.shape[0]

    CH = 2 * NS * SUPER
    E = ((nnz + CH - 1) // CH) * CH
    npad = E - nnz
    pidx = (lax.iota(jnp.int32, npad) * 1009) % N
    rows_p = jnp.concatenate([rows, pidx])
    cols_p = jnp.concatenate([cols, pidx])
    vals_p = jnp.concatenate([vals, jnp.zeros((npad,), jnp.float32)])

    rows2d = rows_p.reshape(E // BLK, BLK)
    cols2d = cols_p.reshape(E // BLK, BLK)
    vals2d = vals_p.reshape(E // BLK, BLK)
    offs = (lax.iota(jnp.int32, 4) * N)
    cols4 = (cols_p[None, :] + offs[:, None]).reshape(4, E // BLK, BLK)

    sA = _stage_a(cols2d, vals2d, E)
    s_part = sA[:, :, 0]
    tab4 = _stage_b(conditions_values, s_part,
                    W_pc1, b_pc1, W_pc2, b_pc2, W_cu1, b_cu1, W_cu2, b_cu2)
    c2v4 = _stage_c(tab4.reshape(4 * N, 16), cols4, rows2d, vals2d, E)
    return _stage_d(c2v4, W_vu1, b_vu1, W_vu2, b_vu2, W_o1, b_o1, W_o2, b_o2)


# speedup vs baseline: 27.0962x; 1.4858x over previous
import functools

import jax
import jax.numpy as jnp
from jax import lax
from jax.experimental import pallas as pl
from jax.experimental.pallas import tpu as pltpu
from jax.experimental.pallas import tpu_sc as plsc

N = 65536
F = 64
NC = 2
NS = 16
BLK = 128
SUB = 8
SUPER = BLK * SUB
ROWS_PER_TILE = N // NS


def _sc_mesh():
    return plsc.VectorSubcoreMesh(
        core_axis_name="c", subcore_axis_name="s", num_cores=NC, num_subcores=NS)


def _stage_a(cols2d, vals2d, E):
    sbt = (E // 2) // SUPER // NS

    @functools.partial(
        pl.kernel,
        out_type=jax.ShapeDtypeStruct((NC, N, 16), jnp.float32),
        mesh=_sc_mesh(),
        scratch_types=[
            pltpu.VMEM_SHARED((N, 16), jnp.float32),
            pltpu.VMEM((SUB, BLK), jnp.int32),
            pltpu.VMEM((SUB, BLK), jnp.float32),
            pltpu.VMEM((SUPER, 16), jnp.float32),
            pltpu.VMEM((SUPER, 16), jnp.float32),
            pltpu.VMEM((BLK, 16), jnp.float32),
            pltpu.SemaphoreType.DMA,
        ],
        compiler_params=pltpu.CompilerParams(use_tc_tiling_on_sc=False),
    )
    def k(cols_hbm, vals_hbm, out_hbm, acc, cbuf, vbuf, obuf, sbuf, zbuf, ssem):
        c = lax.axis_index("c")
        sid = lax.axis_index("s")
        zero16 = jnp.zeros((16,), jnp.float32)
        one16 = jnp.ones((16,), jnp.float32)

        def ob(i, _):
            obuf[i, :] = one16
            return 0
        lax.fori_loop(0, SUPER, ob, 0)

        def zb(i, _):
            zbuf[i, :] = zero16
            return 0
        lax.fori_loop(0, BLK, zb, 0)

        def zacc(t, _):
            off = pl.multiple_of(sid * ROWS_PER_TILE + t * BLK, 8)
            pltpu.sync_copy(zbuf, acc.at[pl.ds(off, BLK)])
            return 0
        lax.fori_loop(0, ROWS_PER_TILE // BLK, zacc, 0)
        plsc.subcore_barrier()

        base = (c * (E // 2) + sid * sbt * SUPER) // BLK

        def body(sb, _):
            rb = pl.multiple_of(base + sb * SUB, 8)
            pltpu.sync_copy(cols_hbm.at[pl.ds(rb, SUB)], cbuf)
            pltpu.sync_copy(vals_hbm.at[pl.ds(rb, SUB)], vbuf)
            for k8 in range(SUB):
                def bcast(j8, _, k8=k8):
                    va16 = jnp.abs(vbuf[k8, pl.ds(j8 * 16, 16)])
                    e0 = k8 * BLK + j8 * 16
                    for i in range(16):
                        er = e0 + i
                        sbuf[er, :] = obuf[er, :] * va16[i]
                    return 0
                lax.fori_loop(0, BLK // 16, bcast, 0)
            ds = [pltpu.async_copy(sbuf.at[pl.ds(k8 * BLK, BLK)],
                                   acc.at[cbuf.at[k8]], ssem, add=True)
                  for k8 in range(SUB)]
            for d in ds:
                d.wait()
            return 0
        lax.fori_loop(0, sbt, body, 0)
        plsc.subcore_barrier()
        sl = pl.ds(pl.multiple_of(sid * ROWS_PER_TILE, 8), ROWS_PER_TILE)
        pltpu.sync_copy(acc.at[sl], out_hbm.at[c, sl])

    return k(cols2d, vals2d)


def _stage_b(cond, s_part, W_pc1, b_pc1, W_pc2, b_pc2, W_cu1, b_cu1, W_cu2, b_cu2):
    BN = 4096
    grid = N // BN

    def body(cond_ref, s0_ref, s1_ref, wp1, bp1, wp2, bp2, wc1, bc1, wc2, bc2, o_ref):
        cond_b = cond_ref[...]
        s = s0_ref[...] + s1_ref[...]
        h1 = jnp.maximum(cond_b * wp1[...] + bp1[...][None, :], 0.0)
        c0 = jnp.dot(h1, wp2[...], preferred_element_type=jnp.float32) + bp2[...][None, :]
        wc1v = wc1[...]
        wsum = jnp.sum(wc1v[F:, :], axis=0)
        h2 = jnp.maximum(
            jnp.dot(c0, wc1v[:F, :], preferred_element_type=jnp.float32)
            + s * wsum[None, :] + bc1[...][None, :], 0.0)
        cc = jnp.dot(h2, wc2[...], preferred_element_type=jnp.float32) + bc2[...][None, :]
        for g in range(4):
            o_ref[g] = cc[:, g * 16:(g + 1) * 16]

    full = lambda shp: pl.BlockSpec(shp, lambda i: tuple(0 for _ in shp))
    return pl.pallas_call(
        body,
        grid=(grid,),
        in_specs=[
            pl.BlockSpec((BN, 1), lambda i: (i, 0)),
            pl.BlockSpec((BN, 1), lambda i: (i, 0)),
            pl.BlockSpec((BN, 1), lambda i: (i, 0)),
            full((1, 2 * F)), full((2 * F,)),
            full((2 * F, F)), full((F,)),
            full((2 * F, 2 * F)), full((2 * F,)),
            full((2 * F, F)), full((F,)),
        ],
        out_specs=pl.BlockSpec((4, BN, 16), lambda i: (0, i, 0)),
        out_shape=jax.ShapeDtypeStruct((4, N, 16), jnp.float32),
    )(cond.reshape(N, 1), s_part[0].reshape(N, 1), s_part[1].reshape(N, 1),
      W_pc1, b_pc1, W_pc2, b_pc2, W_cu1, b_cu1, W_cu2, b_cu2)


def _stage_c(tab_flat, cols4, rows2d, vals2d, E):
    sbt = E // SUPER // NS

    @functools.partial(
        pl.kernel,
        out_type=jax.ShapeDtypeStruct((4, N, 16), jnp.float32),
        mesh=_sc_mesh(),
        scratch_types=[
            pltpu.VMEM_SHARED((N, 16), jnp.float32),
            pltpu.VMEM((2, SUB, BLK), jnp.int32),
            pltpu.VMEM((2, SUB, BLK), jnp.int32),
            pltpu.VMEM((2, SUB, BLK), jnp.float32),
            pltpu.VMEM((2, SUPER, 16), jnp.float32),
            pltpu.VMEM((BLK, 16), jnp.float32),
            pltpu.SemaphoreType.DMA,
            pltpu.SemaphoreType.DMA,
            pltpu.SemaphoreType.DMA,
            pltpu.SemaphoreType.DMA,
        ],
        compiler_params=pltpu.CompilerParams(use_tc_tiling_on_sc=False),
    )
    def k(tab_hbm, cols_hbm, rows_hbm, vals_hbm, out_hbm,
          acc, rbuf, cbuf, vbuf, gbuf, zbuf, lsem, gsem0, gsem1, ssem):
        c = lax.axis_index("c")
        sid = lax.axis_index("s")
        zero16 = jnp.zeros((16,), jnp.float32)

        def zb(i, _):
            zbuf[i, :] = zero16
            return 0
        lax.fori_loop(0, BLK, zb, 0)
        acc_slice = pl.ds(pl.multiple_of(sid * ROWS_PER_TILE, 8), ROWS_PER_TILE)
        base = sid * sbt * SUB

        for p in range(2):
            g = NC * c + p

            def zacc(t, _):
                off = pl.multiple_of(sid * ROWS_PER_TILE + t * BLK, 8)
                pltpu.sync_copy(zbuf, acc.at[pl.ds(off, BLK)])
                return 0
            lax.fori_loop(0, ROWS_PER_TILE // BLK, zacc, 0)
            plsc.subcore_barrier()

            def body(t, _):
                lds = []
                for par in range(2):
                    rb = pl.multiple_of(base + (2 * t + par) * SUB, 8)
                    lds += [
                        pltpu.async_copy(rows_hbm.at[pl.ds(rb, SUB)],
                                         rbuf.at[par], lsem),
                        pltpu.async_copy(cols_hbm.at[g, pl.ds(rb, SUB)],
                                         cbuf.at[par], lsem),
                        pltpu.async_copy(vals_hbm.at[pl.ds(rb, SUB)],
                                         vbuf.at[par], lsem),
                    ]
                for d in lds:
                    d.wait()

                gsems = (gsem0, gsem1)
                gds = [[pltpu.async_copy(tab_hbm.at[cbuf.at[par, k8]],
                                         gbuf.at[par, pl.ds(k8 * BLK, BLK)],
                                         gsems[par])
                        for k8 in range(SUB)] for par in range(2)]

                sds = []
                for par in range(2):
                    for d in gds[par]:
                        d.wait()
                    for k8 in range(SUB):
                        def scale(j8, _, k8=k8, par=par):
                            va16 = jnp.abs(vbuf[par, k8, pl.ds(j8 * 16, 16)])
                            e0 = k8 * BLK + j8 * 16
                            for i in range(16):
                                er = e0 + i
                                gbuf[par, er, :] = gbuf[par, er, :] * va16[i]
                            return 0
                        lax.fori_loop(0, BLK // 16, scale, 0)
                    sds += [pltpu.async_copy(gbuf.at[par, pl.ds(k8 * BLK, BLK)],
                                             acc.at[rbuf.at[par, k8]], ssem,
                                             add=True)
                            for k8 in range(SUB)]
                for d in sds:
                    d.wait()
                return 0
            lax.fori_loop(0, sbt // 2, body, 0)
            plsc.subcore_barrier()
            pltpu.sync_copy(acc.at[acc_slice], out_hbm.at[g, acc_slice])
            plsc.subcore_barrier()

    return k(tab_flat, cols4, rows2d, vals2d)


def _stage_d(c2v4, W_vu1, b_vu1, W_vu2, b_vu2, W_o1, b_o1, W_o2, b_o2):
    BN = 4096
    grid = N // BN

    def body(g0, g1, g2, g3, wv1, bv1, wv2, bv2, wo1, bo1, wo2, bo2, o_ref):
        x = jnp.concatenate([g0[0], g1[0], g2[0], g3[0]], axis=-1)
        wv1v = wv1[...]
        bias1 = bv1[...] + jnp.sum(wv1v[:F, :], axis=0)
        h = jnp.maximum(jnp.dot(x, wv1v[F:, :], preferred_element_type=jnp.float32)
                        + bias1[None, :], 0.0)
        v = jnp.dot(h, wv2[...], preferred_element_type=jnp.float32) + bv2[...][None, :]
        h3 = jnp.maximum(jnp.dot(v, wo1[...], preferred_element_type=jnp.float32)
                         + bo1[...][None, :], 0.0)
        logit = jnp.dot(h3, wo2[...], preferred_element_type=jnp.float32) + bo2[...][None, :]
        o_ref[...] = 1.0 / (1.0 + jnp.exp(-logit))

    full = lambda shp: pl.BlockSpec(shp, lambda i: tuple(0 for _ in shp))
    gspec = lambda g: pl.BlockSpec((1, BN, 16), lambda i, g=g: (g, i, 0))
    return pl.pallas_call(
        body,
        grid=(grid,),
        in_specs=[
            gspec(0), gspec(1), gspec(2), gspec(3),
            full((2 * F, 2 * F)), full((2 * F,)),
            full((2 * F, F)), full((F,)),
            full((F, 2 * F)), full((2 * F,)),
            full((2 * F, 1)), full((1,)),
        ],
        out_specs=pl.BlockSpec((BN, 1), lambda i: (i, 0)),
        out_shape=jax.ShapeDtypeStruct((N, 1), jnp.float32),
    )(c2v4, c2v4, c2v4, c2v4, W_vu1, b_vu1, W_vu2, b_vu2, W_o1, b_o1, W_o2, b_o2)


def kernel(adj_indices, adj_values, conditions_values,
           W_pc1, b_pc1, W_pc2, b_pc2, W_cu1, b_cu1, W_cu2, b_cu2,
           W_vu1, b_vu1, W_vu2, b_vu2, W_o1, b_o1, W_o2, b_o2):
    rows = adj_indices[0].astype(jnp.int32)
    cols = adj_indices[1].astype(jnp.int32)
    vals = adj_values.astype(jnp.float32)
    nnz = rows.shape[0]

    CH = 2 * NS * SUPER
    E = ((nnz + CH - 1) // CH) * CH
    npad = E - nnz
    pidx = (lax.iota(jnp.int32, npad) * 1009) % N
    rows_p = jnp.concatenate([rows, pidx])
    cols_p = jnp.concatenate([cols, pidx])
    vals_p = jnp.concatenate([vals, jnp.zeros((npad,), jnp.float32)])

    rows2d = rows_p.reshape(E // BLK, BLK)
    cols2d = cols_p.reshape(E // BLK, BLK)
    vals2d = vals_p.reshape(E // BLK, BLK)
    offs = (lax.iota(jnp.int32, 4) * N)
    cols4 = (cols_p[None, :] + offs[:, None]).reshape(4, E // BLK, BLK)

    sA = _stage_a(cols2d, vals2d, E)
    s_part = sA[:, :, 0]
    tab4 = _stage_b(conditions_values, s_part,
                    W_pc1, b_pc1, W_pc2, b_pc2, W_cu1, b_cu1, W_cu2, b_cu2)
    c2v4 = _stage_c(tab4.reshape(4 * N, 16), cols4, rows2d, vals2d, E)
    return _stage_d(c2v4, W_vu1, b_vu1, W_vu2, b_vu2, W_o1, b_o1, W_o2, b_o2)
